# baseline (speedup 1.0000x reference)
import jax
import jax.numpy as jnp
from jax.experimental import pallas as pl
from jax.experimental.pallas import tpu as pltpu

N = 4096
D = 512
BM = 256
NB = N // BM


def _pair_body(emb_ref, w1t_ref, w2t_ref, bfc1_ref, bfc2_ref, bias1_ref,
               bias2_ref, p1_ref, p2_ref, wslat_ref, bsla_ref, asla_ref,
               meta1_ref, meta2_ref, out_ref,
               h_scr, v1_scr, v2_scr, acc1_scr, acc2_scr):
    i = pl.program_id(0)

    @pl.when(i == 0)
    def _h1():
        h = jnp.dot(emb_ref[...], w1t_ref[...],
                    preferred_element_type=jnp.float32) + bfc1_ref[...]
        h_scr[...] = h.astype(jnp.bfloat16)
        acc1_scr[...] = jnp.zeros_like(acc1_scr)

    @pl.when(i == NB)
    def _h2():
        h = jnp.dot(emb_ref[...], w2t_ref[...],
                    preferred_element_type=jnp.float32) + bfc2_ref[...]
        h_scr[...] = h.astype(jnp.bfloat16)
        acc2_scr[...] = jnp.zeros_like(acc2_scr)

    @pl.when(i < NB)
    def _spmm1():
        out = jnp.dot(meta1_ref[...].astype(jnp.bfloat16), h_scr[...],
                      preferred_element_type=jnp.float32) + bias1_ref[...]
        vb = jnp.where(out >= 0, out, p1_ref[0, 0] * out).astype(jnp.bfloat16)
        v1_scr[pl.ds(i * BM, BM), :] = vb

    @pl.when(jnp.logical_and(i >= NB, i < 2 * NB))
    def _spmm2():
        j = i - NB
        out = jnp.dot(meta2_ref[...].astype(jnp.bfloat16), h_scr[...],
                      preferred_element_type=jnp.float32) + bias2_ref[...]
        vb = jnp.where(out >= 0, out, p2_ref[0, 0] * out).astype(jnp.bfloat16)
        v2_scr[pl.ds(j * BM, BM), :] = vb

    @pl.when(jnp.logical_and(i >= 1, i <= NB))
    def _feat1():
        vb = v1_scr[pl.ds((i - 1) * BM, BM), :]
        s = jnp.tanh(jnp.dot(vb, wslat_ref[...],
                             preferred_element_type=jnp.float32) + bsla_ref[...])
        acc1_scr[...] += jnp.sum(s, axis=0, keepdims=True)

    @pl.when(jnp.logical_and(i >= NB + 1, i <= 2 * NB))
    def _feat2():
        vb = v2_scr[pl.ds((i - NB - 1) * BM, BM), :]
        s = jnp.tanh(jnp.dot(vb, wslat_ref[...],
                             preferred_element_type=jnp.float32) + bsla_ref[...])
        acc2_scr[...] += jnp.sum(s, axis=0, keepdims=True)

    @pl.when(i >= 2 * NB)
    def _combine():
        j = i - 2 * NB
        la = jnp.sum(asla_ref[...] * acc1_scr[...] * (1.0 / N),
                     axis=1, keepdims=True)
        lb = jnp.sum(asla_ref[...] * acc2_scr[...] * (1.0 / N),
                     axis=1, keepdims=True)
        m = jnp.maximum(la, lb)
        ea = jnp.exp(la - m)
        eb = jnp.exp(lb - m)
        inv = 1.0 / (ea + eb)
        b1 = ea * inv
        b2 = eb * inv
        v1 = v1_scr[pl.ds(j * BM, BM), :].astype(jnp.float32)
        v2 = v2_scr[pl.ds(j * BM, BM), :].astype(jnp.float32)
        out_ref[...] = v1 * b1 + v2 * b2


def _pair_call(emb_bf, w1t, w2t, bfc1, bfc2, bias1, bias2, p1, p2,
               wslat, bsla, asla, meta1, meta2):
    const = lambda i: (0, 0)
    return pl.pallas_call(
        _pair_body,
        grid=(3 * NB,),
        in_specs=[
            pl.BlockSpec((N, D), const),
            pl.BlockSpec((D, D), const),
            pl.BlockSpec((D, D), const),
            pl.BlockSpec((1, D), const),
            pl.BlockSpec((1, D), const),
            pl.BlockSpec((1, D), const),
            pl.BlockSpec((1, D), const),
            pl.BlockSpec((1, 1), const),
            pl.BlockSpec((1, 1), const),
            pl.BlockSpec((D, D), const),
            pl.BlockSpec((1, D), const),
            pl.BlockSpec((1, D), const),
            pl.BlockSpec((BM, N), lambda i: (jnp.minimum(i, NB - 1), 0)),
            pl.BlockSpec((BM, N),
                         lambda i: (jnp.clip(i - NB, 0, NB - 1), 0)),
        ],
        out_specs=pl.BlockSpec((BM, D),
                               lambda i: (jnp.clip(i - 2 * NB, 0, NB - 1), 0)),
        out_shape=jax.ShapeDtypeStruct((N, D), jnp.float32),
        scratch_shapes=[
            pltpu.VMEM((N, D), jnp.bfloat16),
            pltpu.VMEM((N, D), jnp.bfloat16),
            pltpu.VMEM((N, D), jnp.bfloat16),
            pltpu.VMEM((1, D), jnp.float32),
            pltpu.VMEM((1, D), jnp.float32),
        ],
    )(emb_bf, w1t, w2t, bfc1, bfc2, bias1, bias2, p1, p2, wslat, bsla, asla,
      meta1, meta2)


@jax.jit
def kernel(emb_mi, emb_di, meta_mdm, meta_mdmdm, meta_dmd, meta_dmdmd,
           W_mdm, bfc_mdm, bias_mdm, p_mdm,
           W_mdmdm, bfc_mdmdm, bias_mdmdm, p_mdmdm,
           W_dmd, bfc_dmd, bias_dmd, p_dmd,
           W_dmdmd, bfc_dmdmd, bias_dmdmd, p_dmdmd,
           W_sla, b_sla, a_sla):
    wslat = W_sla.T.astype(jnp.bfloat16)
    bsla = b_sla.reshape(1, D)
    asla = a_sla.reshape(1, D)

    out_mi = _pair_call(
        emb_mi.astype(jnp.bfloat16),
        W_mdm.T.astype(jnp.bfloat16), W_mdmdm.T.astype(jnp.bfloat16),
        bfc_mdm.reshape(1, D), bfc_mdmdm.reshape(1, D),
        bias_mdm.reshape(1, D), bias_mdmdm.reshape(1, D),
        p_mdm.reshape(1, 1), p_mdmdm.reshape(1, 1),
        wslat, bsla, asla, meta_mdm, meta_mdmdm)
    out_di = _pair_call(
        emb_di.astype(jnp.bfloat16),
        W_dmd.T.astype(jnp.bfloat16), W_dmdmd.T.astype(jnp.bfloat16),
        bfc_dmd.reshape(1, D), bfc_dmdmd.reshape(1, D),
        bias_dmd.reshape(1, D), bias_dmdmd.reshape(1, D),
        p_dmd.reshape(1, 1), p_dmdmd.reshape(1, 1),
        wslat, bsla, asla, meta_dmd, meta_dmdmd)
    return out_mi, out_di


# speedup vs baseline: 1.0293x; 1.0293x over previous
import jax
import jax.numpy as jnp
from jax.experimental import pallas as pl
from jax.experimental.pallas import tpu as pltpu

N = 4096
D = 512
BM = 256
NB = N // BM


def _feat(vb, wslat_ref, bsla_ref):
    s = jnp.tanh(jnp.dot(vb, wslat_ref[...],
                         preferred_element_type=jnp.float32) + bsla_ref[...])
    return jnp.sum(s, axis=0, keepdims=True)


def _pair_body(emb_ref, w1t_ref, w2t_ref, bfc1_ref, bfc2_ref, bias1_ref,
               bias2_ref, p1_ref, p2_ref, wslat_ref, bsla_ref, asla_ref,
               meta1_ref, meta2_ref, out_ref,
               h_scr, v1_scr, v2_scr, acc1_scr, acc2_scr):
    i = pl.program_id(0)

    @pl.when(i == 0)
    def _h1():
        h = jnp.dot(emb_ref[...], w1t_ref[...],
                    preferred_element_type=jnp.float32) + bfc1_ref[...]
        h_scr[...] = h.astype(jnp.bfloat16)
        acc1_scr[...] = jnp.zeros_like(acc1_scr)

    @pl.when(i < NB)
    def _phase1():
        prev = jnp.maximum(i - 1, 0)
        vp = v1_scr[pl.ds(prev * BM, BM), :]
        out = jnp.dot(meta1_ref[...].astype(jnp.bfloat16), h_scr[...],
                      preferred_element_type=jnp.float32) + bias1_ref[...]
        vb = jnp.where(out >= 0, out, p1_ref[0, 0] * out).astype(jnp.bfloat16)
        cs = _feat(vp, wslat_ref, bsla_ref)
        acc1_scr[...] += jnp.where(i >= 1, cs, jnp.zeros_like(cs))
        v1_scr[pl.ds(i * BM, BM), :] = vb

    @pl.when(i == NB)
    def _h2():
        h = jnp.dot(emb_ref[...], w2t_ref[...],
                    preferred_element_type=jnp.float32) + bfc2_ref[...]
        h_scr[...] = h.astype(jnp.bfloat16)
        acc1_scr[...] += _feat(v1_scr[pl.ds((NB - 1) * BM, BM), :],
                               wslat_ref, bsla_ref)
        acc2_scr[...] = jnp.zeros_like(acc2_scr)

    @pl.when(jnp.logical_and(i >= NB, i < 2 * NB))
    def _phase2():
        j = i - NB
        prev = jnp.maximum(j - 1, 0)
        vp = v2_scr[pl.ds(prev * BM, BM), :]
        out = jnp.dot(meta2_ref[...].astype(jnp.bfloat16), h_scr[...],
                      preferred_element_type=jnp.float32) + bias2_ref[...]
        vb = jnp.where(out >= 0, out, p2_ref[0, 0] * out).astype(jnp.bfloat16)
        cs = _feat(vp, wslat_ref, bsla_ref)
        acc2_scr[...] += jnp.where(j >= 1, cs, jnp.zeros_like(cs))
        v2_scr[pl.ds(j * BM, BM), :] = vb

    @pl.when(i == 2 * NB)
    def _feat2_tail():
        acc2_scr[...] += _feat(v2_scr[pl.ds((NB - 1) * BM, BM), :],
                               wslat_ref, bsla_ref)

    @pl.when(i >= 2 * NB)
    def _combine():
        j = i - 2 * NB
        la = jnp.sum(asla_ref[...] * acc1_scr[...] * (1.0 / N),
                     axis=1, keepdims=True)
        lb = jnp.sum(asla_ref[...] * acc2_scr[...] * (1.0 / N),
                     axis=1, keepdims=True)
        m = jnp.maximum(la, lb)
        ea = jnp.exp(la - m)
        eb = jnp.exp(lb - m)
        inv = 1.0 / (ea + eb)
        b1 = ea * inv
        b2 = eb * inv
        v1 = v1_scr[pl.ds(j * BM, BM), :].astype(jnp.float32)
        v2 = v2_scr[pl.ds(j * BM, BM), :].astype(jnp.float32)
        out_ref[...] = v1 * b1 + v2 * b2


def _pair_call(emb_bf, w1t, w2t, bfc1, bfc2, bias1, bias2, p1, p2,
               wslat, bsla, asla, meta1, meta2):
    const = lambda i: (0, 0)
    return pl.pallas_call(
        _pair_body,
        grid=(3 * NB,),
        in_specs=[
            pl.BlockSpec((N, D), const),
            pl.BlockSpec((D, D), const),
            pl.BlockSpec((D, D), const),
            pl.BlockSpec((1, D), const),
            pl.BlockSpec((1, D), const),
            pl.BlockSpec((1, D), const),
            pl.BlockSpec((1, D), const),
            pl.BlockSpec((1, 1), const),
            pl.BlockSpec((1, 1), const),
            pl.BlockSpec((D, D), const),
            pl.BlockSpec((1, D), const),
            pl.BlockSpec((1, D), const),
            pl.BlockSpec((BM, N), lambda i: (jnp.minimum(i, NB - 1), 0)),
            pl.BlockSpec((BM, N),
                         lambda i: (jnp.clip(i - NB, 0, NB - 1), 0)),
        ],
        out_specs=pl.BlockSpec((BM, D),
                               lambda i: (jnp.clip(i - 2 * NB, 0, NB - 1), 0)),
        out_shape=jax.ShapeDtypeStruct((N, D), jnp.float32),
        scratch_shapes=[
            pltpu.VMEM((N, D), jnp.bfloat16),
            pltpu.VMEM((N, D), jnp.bfloat16),
            pltpu.VMEM((N, D), jnp.bfloat16),
            pltpu.VMEM((1, D), jnp.float32),
            pltpu.VMEM((1, D), jnp.float32),
        ],
    )(emb_bf, w1t, w2t, bfc1, bfc2, bias1, bias2, p1, p2, wslat, bsla, asla,
      meta1, meta2)


@jax.jit
def kernel(emb_mi, emb_di, meta_mdm, meta_mdmdm, meta_dmd, meta_dmdmd,
           W_mdm, bfc_mdm, bias_mdm, p_mdm,
           W_mdmdm, bfc_mdmdm, bias_mdmdm, p_mdmdm,
           W_dmd, bfc_dmd, bias_dmd, p_dmd,
           W_dmdmd, bfc_dmdmd, bias_dmdmd, p_dmdmd,
           W_sla, b_sla, a_sla):
    wslat = W_sla.T.astype(jnp.bfloat16)
    bsla = b_sla.reshape(1, D)
    asla = a_sla.reshape(1, D)

    out_mi = _pair_call(
        emb_mi.astype(jnp.bfloat16),
        W_mdm.T.astype(jnp.bfloat16), W_mdmdm.T.astype(jnp.bfloat16),
        bfc_mdm.reshape(1, D), bfc_mdmdm.reshape(1, D),
        bias_mdm.reshape(1, D), bias_mdmdm.reshape(1, D),
        p_mdm.reshape(1, 1), p_mdmdm.reshape(1, 1),
        wslat, bsla, asla, meta_mdm, meta_mdmdm)
    out_di = _pair_call(
        emb_di.astype(jnp.bfloat16),
        W_dmd.T.astype(jnp.bfloat16), W_dmdmd.T.astype(jnp.bfloat16),
        bfc_dmd.reshape(1, D), bfc_dmdmd.reshape(1, D),
        bias_dmd.reshape(1, D), bias_dmdmd.reshape(1, D),
        p_dmd.reshape(1, 1), p_dmdmd.reshape(1, 1),
        wslat, bsla, asla, meta_dmd, meta_dmdmd)
    return out_mi, out_di


# speedup vs baseline: 1.1504x; 1.1177x over previous
import jax
import jax.numpy as jnp
from jax.experimental import pallas as pl
from jax.experimental.pallas import tpu as pltpu

N = 4096
D = 512
BM = 256
NB = N // BM


def _feat(vb, wslat_ref, bsla_ref):
    s = jnp.tanh(jnp.dot(vb, wslat_ref[...],
                         preferred_element_type=jnp.float32) + bsla_ref[...])
    return jnp.sum(s, axis=0, keepdims=True)


def _pair_body(emb_ref, w1t_ref, w2t_ref, bfc1_ref, bfc2_ref, bias1_ref,
               bias2_ref, p1_ref, p2_ref, wslat_ref, bsla_ref, asla_ref,
               meta1_ref, meta2_ref, out_ref,
               h1_scr, h2_scr, v1_scr, v2_scr, acc1_scr, acc2_scr):
    i = pl.program_id(0)

    @pl.when(i == 0)
    def _init():
        h1 = jnp.dot(emb_ref[...], w1t_ref[...],
                     preferred_element_type=jnp.float32) + bfc1_ref[...]
        h1_scr[...] = h1.astype(jnp.bfloat16)
        h2 = jnp.dot(emb_ref[...], w2t_ref[...],
                     preferred_element_type=jnp.float32) + bfc2_ref[...]
        h2_scr[...] = h2.astype(jnp.bfloat16)
        acc1_scr[...] = jnp.zeros_like(acc1_scr)
        acc2_scr[...] = jnp.zeros_like(acc2_scr)

    @pl.when(i < NB)
    def _spmm():
        prev = jnp.maximum(i - 1, 0)
        vp1 = v1_scr[pl.ds(prev * BM, BM), :]
        vp2 = v2_scr[pl.ds(prev * BM, BM), :]
        out1 = jnp.dot(meta1_ref[...].astype(jnp.bfloat16), h1_scr[...],
                       preferred_element_type=jnp.float32) + bias1_ref[...]
        vb1 = jnp.where(out1 >= 0, out1,
                        p1_ref[0, 0] * out1).astype(jnp.bfloat16)
        out2 = jnp.dot(meta2_ref[...].astype(jnp.bfloat16), h2_scr[...],
                       preferred_element_type=jnp.float32) + bias2_ref[...]
        vb2 = jnp.where(out2 >= 0, out2,
                        p2_ref[0, 0] * out2).astype(jnp.bfloat16)
        cs1 = _feat(vp1, wslat_ref, bsla_ref)
        cs2 = _feat(vp2, wslat_ref, bsla_ref)
        acc1_scr[...] += jnp.where(i >= 1, cs1, jnp.zeros_like(cs1))
        acc2_scr[...] += jnp.where(i >= 1, cs2, jnp.zeros_like(cs2))
        v1_scr[pl.ds(i * BM, BM), :] = vb1
        v2_scr[pl.ds(i * BM, BM), :] = vb2

    @pl.when(i == NB)
    def _feat_tail():
        acc1_scr[...] += _feat(v1_scr[pl.ds((NB - 1) * BM, BM), :],
                               wslat_ref, bsla_ref)
        acc2_scr[...] += _feat(v2_scr[pl.ds((NB - 1) * BM, BM), :],
                               wslat_ref, bsla_ref)

    @pl.when(i >= NB)
    def _combine():
        j = i - NB
        la = jnp.sum(asla_ref[...] * acc1_scr[...] * (1.0 / N),
                     axis=1, keepdims=True)
        lb = jnp.sum(asla_ref[...] * acc2_scr[...] * (1.0 / N),
                     axis=1, keepdims=True)
        m = jnp.maximum(la, lb)
        ea = jnp.exp(la - m)
        eb = jnp.exp(lb - m)
        inv = 1.0 / (ea + eb)
        b1 = ea * inv
        b2 = eb * inv
        v1 = v1_scr[pl.ds(j * BM, BM), :].astype(jnp.float32)
        v2 = v2_scr[pl.ds(j * BM, BM), :].astype(jnp.float32)
        out_ref[...] = v1 * b1 + v2 * b2


def _pair_call(emb_bf, w1t, w2t, bfc1, bfc2, bias1, bias2, p1, p2,
               wslat, bsla, asla, meta1, meta2):
    const = lambda i: (0, 0)
    mblk = pl.BlockSpec((BM, N), lambda i: (jnp.minimum(i, NB - 1), 0))
    return pl.pallas_call(
        _pair_body,
        grid=(2 * NB,),
        in_specs=[
            pl.BlockSpec((N, D), const),
            pl.BlockSpec((D, D), const),
            pl.BlockSpec((D, D), const),
            pl.BlockSpec((1, D), const),
            pl.BlockSpec((1, D), const),
            pl.BlockSpec((1, D), const),
            pl.BlockSpec((1, D), const),
            pl.BlockSpec((1, 1), const),
            pl.BlockSpec((1, 1), const),
            pl.BlockSpec((D, D), const),
            pl.BlockSpec((1, D), const),
            pl.BlockSpec((1, D), const),
            mblk,
            mblk,
        ],
        out_specs=pl.BlockSpec((BM, D),
                               lambda i: (jnp.clip(i - NB, 0, NB - 1), 0)),
        out_shape=jax.ShapeDtypeStruct((N, D), jnp.float32),
        scratch_shapes=[
            pltpu.VMEM((N, D), jnp.bfloat16),
            pltpu.VMEM((N, D), jnp.bfloat16),
            pltpu.VMEM((N, D), jnp.bfloat16),
            pltpu.VMEM((N, D), jnp.bfloat16),
            pltpu.VMEM((1, D), jnp.float32),
            pltpu.VMEM((1, D), jnp.float32),
        ],
    )(emb_bf, w1t, w2t, bfc1, bfc2, bias1, bias2, p1, p2, wslat, bsla, asla,
      meta1, meta2)


@jax.jit
def kernel(emb_mi, emb_di, meta_mdm, meta_mdmdm, meta_dmd, meta_dmdmd,
           W_mdm, bfc_mdm, bias_mdm, p_mdm,
           W_mdmdm, bfc_mdmdm, bias_mdmdm, p_mdmdm,
           W_dmd, bfc_dmd, bias_dmd, p_dmd,
           W_dmdmd, bfc_dmdmd, bias_dmdmd, p_dmdmd,
           W_sla, b_sla, a_sla):
    wslat = W_sla.T.astype(jnp.bfloat16)
    bsla = b_sla.reshape(1, D)
    asla = a_sla.reshape(1, D)

    out_mi = _pair_call(
        emb_mi.astype(jnp.bfloat16),
        W_mdm.T.astype(jnp.bfloat16), W_mdmdm.T.astype(jnp.bfloat16),
        bfc_mdm.reshape(1, D), bfc_mdmdm.reshape(1, D),
        bias_mdm.reshape(1, D), bias_mdmdm.reshape(1, D),
        p_mdm.reshape(1, 1), p_mdmdm.reshape(1, 1),
        wslat, bsla, asla, meta_mdm, meta_mdmdm)
    out_di = _pair_call(
        emb_di.astype(jnp.bfloat16),
        W_dmd.T.astype(jnp.bfloat16), W_dmdmd.T.astype(jnp.bfloat16),
        bfc_dmd.reshape(1, D), bfc_dmdmd.reshape(1, D),
        bias_dmd.reshape(1, D), bias_dmdmd.reshape(1, D),
        p_dmd.reshape(1, 1), p_dmdmd.reshape(1, 1),
        wslat, bsla, asla, meta_dmd, meta_dmdmd)
    return out_mi, out_di
